# baseline (speedup 1.0000x reference)
import functools

import numpy as np
import jax
import jax.numpy as jnp
from jax import lax
from jax.experimental import pallas as pl
from jax.experimental.pallas import tpu as pltpu
from jax.experimental.pallas import tpu_sc as plsc

_D = 64
_SEQ = 16384
_NC = 2
_NS = 16
_L = 16
_NW = _NC * _NS
_BPW = _SEQ // _NW
_CHUNK = 128
_NCHUNK = _BPW // _CHUNK
_RUNROLL = 4


def _pe_table() -> np.ndarray:
    i = np.arange(_SEQ, dtype=np.float32)[:, None]
    j = np.arange(_D, dtype=np.float32)[None, :]
    angle = i / np.power(np.float32(10000.0), j / np.float32(_D))
    even = (np.arange(_D)[None, :] % 2) == 0
    return np.where(even, np.sin(angle), np.cos(angle)).astype(np.float32)


_PE = _pe_table()


def _sc_embed(table, x, pe):
    mesh = plsc.VectorSubcoreMesh(core_axis_name="c", subcore_axis_name="s")

    @functools.partial(
        pl.kernel,
        out_type=jax.ShapeDtypeStruct((_SEQ, _D), jnp.float32),
        mesh=mesh,
        scratch_types=[
            pltpu.VMEM((_BPW,), jnp.int32),
            pltpu.VMEM((_BPW, _D), jnp.float32),
            pltpu.VMEM((_BPW, _D), jnp.float32),
            pltpu.SemaphoreType.DMA,
        ],
        compiler_params=pltpu.CompilerParams(use_tc_tiling_on_sc=False),
    )
    def k(table_hbm, x_hbm, pe_hbm, out_hbm, idx_v, rows_v, pe_v, sem):
        wid = lax.axis_index("s") * _NC + lax.axis_index("c")
        base = wid * _BPW
        pltpu.sync_copy(x_hbm.at[pl.ds(base, _BPW)], idx_v)
        gathers = []
        for c in range(_NCHUNK):
            gathers.append(pltpu.async_copy(
                table_hbm.at[idx_v.at[pl.ds(c * _CHUNK, _CHUNK)]],
                rows_v.at[pl.ds(c * _CHUNK, _CHUNK)],
                sem,
            ))
        pltpu.sync_copy(pe_hbm.at[pl.ds(base, _BPW)], pe_v)
        for g in gathers:
            g.wait()

        @pl.loop(0, _BPW, step=_RUNROLL)
        def _(r):
            for dr in range(_RUNROLL):
                for c in range(_D // _L):
                    slc = (r + dr, pl.ds(c * _L, _L))
                    rows_v.at[slc][...] = rows_v.at[slc][...] + pe_v.at[slc][...]

        pltpu.sync_copy(rows_v, out_hbm.at[pl.ds(base, _BPW)])

    return k(table, x, pe)


def kernel(x, table):
    return _sc_embed(table, x, _PE)
